# baseline (speedup 1.0000x reference)
import jax
import jax.numpy as jnp
from jax import lax
from jax.experimental import pallas as pl
from jax.experimental.pallas import tpu as pltpu
from jax.experimental.pallas import tpu_sc as plsc

_B = 16384
_D = 32
_V = 1000000
_NC = 2
_NS = 16
_NW = _NC * _NS
_BPW = _B // _NW
_EPW = _BPW * _D
_NDESC = 16
_EPD = _EPW // _NDESC


def _emb_body(table_hbm, idx_hbm, out_hbm, idx_v, offs_v, rows_v, sem):
    wid = lax.axis_index("s") * _NC + lax.axis_index("c")
    base = wid * _BPW
    pltpu.sync_copy(idx_hbm.at[pl.ds(base, _BPW)], idx_v)

    iota = lax.iota(jnp.int32, 16)

    def block(b, carry):
        iv = idx_v[pl.ds(b * 16, 16)] * _D
        for i in range(16):
            j = b * 16 + i
            lo = jnp.full((16,), iv[i], jnp.int32) + iota
            d = j // (_EPD // _D)
            col = (j % (_EPD // _D)) * _D
            offs_v[d, pl.ds(col, 16)] = lo
            offs_v[d, pl.ds(col + 16, 16)] = lo + 16
        return carry

    lax.fori_loop(0, _BPW // 16, block, 0)

    handles = [
        pltpu.async_copy(table_hbm.at[offs_v.at[d]],
                         rows_v.at[pl.ds(d * _EPD, _EPD)], sem)
        for d in range(_NDESC)
    ]
    for h in handles:
        h.wait()

    pltpu.sync_copy(rows_v, out_hbm.at[pl.ds(base * _D, _EPW)])


@jax.jit
def _lookup(table1, idx):
    mesh = plsc.VectorSubcoreMesh(core_axis_name="c", subcore_axis_name="s")
    f = pl.kernel(
        _emb_body,
        out_type=jax.ShapeDtypeStruct((_B * _D,), jnp.float32),
        mesh=mesh,
        compiler_params=pltpu.CompilerParams(use_tc_tiling_on_sc=False),
        scratch_types=[
            pltpu.VMEM((_BPW,), jnp.int32),
            pltpu.VMEM((_NDESC, _EPD), jnp.int32),
            pltpu.VMEM((_EPW,), jnp.float32),
            pltpu.SemaphoreType.DMA,
        ],
    )
    return f(table1, idx)


def kernel(idx, table):
    table1 = table.reshape(-1)
    out = _lookup(table1, idx.astype(jnp.int32))
    return out.reshape(-1, _D, 1, 1)


# speedup vs baseline: 1.6869x; 1.6869x over previous
import jax
import jax.numpy as jnp
from jax import lax
from jax.experimental import pallas as pl
from jax.experimental.pallas import tpu as pltpu
from jax.experimental.pallas import tpu_sc as plsc

_B = 16384
_D = 32
_NC = 2
_NS = 16
_NW = _NC * _NS
_BPW = _B // _NW
_NSEM = 8
_CH = 64
_NCHUNK = _BPW // _CH


def _emb_body(table_hbm, idx_hbm, out_hbm, idx_v, out_v, *sems):
    wid = lax.axis_index("s") * _NC + lax.axis_index("c")
    base = wid * _BPW
    pltpu.sync_copy(idx_hbm.at[pl.ds(base, _BPW)], idx_v)

    def issue_chunk(q):
        hs = []
        for bb in range(_CH // 16):
            iv = idx_v[pl.ds(q * _CH + bb * 16, 16)]
            for i in range(16):
                j = q * _CH + bb * 16 + i
                hs.append(pltpu.async_copy(table_hbm.at[iv[i]],
                                           out_v.at[j], sems[j % _NSEM]))
        return hs

    pending = {}
    for q in range(_NCHUNK):
        pending[q] = issue_chunk(q)
        if q >= 1:
            for h in pending.pop(q - 1):
                h.wait()
    for h in pending.pop(_NCHUNK - 1):
        h.wait()

    pltpu.sync_copy(out_v, out_hbm.at[pl.ds(base, _BPW)])


@jax.jit
def _lookup(table, idx):
    mesh = plsc.VectorSubcoreMesh(core_axis_name="c", subcore_axis_name="s")
    f = pl.kernel(
        _emb_body,
        out_type=jax.ShapeDtypeStruct((_B, _D), jnp.float32),
        mesh=mesh,
        scratch_types=[
            pltpu.VMEM((_BPW,), jnp.int32),
            pltpu.VMEM((_BPW, _D), jnp.float32),
        ] + [pltpu.SemaphoreType.DMA] * _NSEM,
    )
    return f(table, idx)


def kernel(idx, table):
    out = _lookup(table, idx.astype(jnp.int32))
    return out.reshape(-1, _D, 1, 1)
